# baseline (speedup 1.0000x reference)
import jax
import jax.numpy as jnp
from jax import lax
from jax.experimental import pallas as pl
from jax.experimental.pallas import tpu as pltpu

RG = 128
W = 128
SL = 8


def _chamfer_kernel(x_ref, y_ref, d1_ref, i1_ref, d2_ref, i2_ref,
                    cp_ref, cpi_ref):
    n = x_ref.shape[1]
    m = y_ref.shape[2]
    nchunks = m // W
    ngroups = n // RG
    nvr = RG // SL

    big = jnp.float32(2**24)
    inf = jnp.float32(jnp.inf)

    jlane = lax.broadcasted_iota(jnp.int32, (1, W), 1).astype(jnp.float32)
    riota0 = (lax.broadcasted_iota(jnp.int32, (nvr, SL, 1), 0) * SL
              + lax.broadcasted_iota(jnp.int32, (nvr, SL, 1), 1)
              ).astype(jnp.float32)

    cp_ref[...] = jnp.full((SL, m), inf, jnp.float32)
    cpi_ref[...] = jnp.zeros((SL, m), jnp.float32)

    def body(rg, _):
        row0 = rg * RG
        xr = x_ref[0, pl.ds(row0, RG), :]
        x0 = xr[:, 0:1]
        x1 = xr[:, 1:2]
        x2 = xr[:, 2:3]
        riota3 = riota0 + row0.astype(jnp.float32)

        rowbest = jnp.full((RG, W), inf, jnp.float32)
        rowbesti = jnp.zeros((RG, W), jnp.float32)

        for c in range(nchunks):
            lo = c * W
            yc = y_ref[0, :, lo:lo + W]
            t0 = x0 - yc[0:1, :]
            d = t0 * t0
            t1 = x1 - yc[1:2, :]
            d = d + t1 * t1
            t2 = x2 - yc[2:3, :]
            d = d + t2 * t2

            mask = d < rowbest
            rowbest = jnp.where(mask, d, rowbest)
            rowbesti = jnp.where(mask, jlane + jnp.float32(lo), rowbesti)

            d3 = d.reshape(nvr, SL, W)
            cp = jnp.min(d3, axis=0)
            cpi = jnp.min(jnp.where(d3 == cp[None], riota3, big), axis=0)
            prev = cp_ref[:, lo:lo + W]
            previ = cpi_ref[:, lo:lo + W]
            upd = cp < prev
            cp_ref[:, lo:lo + W] = jnp.where(upd, cp, prev)
            cpi_ref[:, lo:lo + W] = jnp.where(upd, cpi, previ)

        m1 = jnp.min(rowbest, axis=1, keepdims=True)
        i1f = jnp.min(jnp.where(rowbest == m1, rowbesti, big), axis=1,
                      keepdims=True)
        d1_ref[0, pl.ds(row0, RG), :] = m1
        i1_ref[0, pl.ds(row0, RG), :] = i1f.astype(jnp.int32)
        return 0

    lax.fori_loop(0, ngroups, body, 0)

    cpf = cp_ref[...]
    cpfi = cpi_ref[...]
    m2 = jnp.min(cpf, axis=0, keepdims=True)
    i2f = jnp.min(jnp.where(cpf == m2, cpfi, big), axis=0, keepdims=True)
    d2_ref[0] = m2
    i2_ref[0] = i2f.astype(jnp.int32)


def kernel(input1, input2):
    b, n, _ = input1.shape
    m = input2.shape[1]
    y_t = input2.transpose(0, 2, 1)

    d1, i1, d2, i2 = pl.pallas_call(
        _chamfer_kernel,
        grid=(b,),
        in_specs=[
            pl.BlockSpec((1, n, 3), lambda bi: (bi, 0, 0)),
            pl.BlockSpec((1, 3, m), lambda bi: (bi, 0, 0)),
        ],
        out_specs=[
            pl.BlockSpec((1, n, 1), lambda bi: (bi, 0, 0)),
            pl.BlockSpec((1, n, 1), lambda bi: (bi, 0, 0)),
            pl.BlockSpec((1, 1, m), lambda bi: (bi, 0, 0)),
            pl.BlockSpec((1, 1, m), lambda bi: (bi, 0, 0)),
        ],
        out_shape=[
            jax.ShapeDtypeStruct((b, n, 1), jnp.float32),
            jax.ShapeDtypeStruct((b, n, 1), jnp.int32),
            jax.ShapeDtypeStruct((b, 1, m), jnp.float32),
            jax.ShapeDtypeStruct((b, 1, m), jnp.int32),
        ],
        scratch_shapes=[
            pltpu.VMEM((SL, m), jnp.float32),
            pltpu.VMEM((SL, m), jnp.float32),
        ],
        compiler_params=pltpu.CompilerParams(
            dimension_semantics=("parallel",)),
    )(input1, y_t)

    dist1 = d1.reshape(b, n)
    idx1 = i1.reshape(b, n)
    dist2 = d2.reshape(b, m)
    idx2 = i2.reshape(b, m)
    return (dist1, dist2, idx1, idx2)


# speedup vs baseline: 1.0640x; 1.0640x over previous
import jax
import jax.numpy as jnp
from jax import lax
from jax.experimental import pallas as pl
from jax.experimental.pallas import tpu as pltpu

NB = 512


def _chamfer_kernel(x_ref, y_ref, d1_ref, i1_ref, d2_ref, i2_ref):
    ni = pl.program_id(1)
    x = x_ref[0]
    y = y_ref[0]
    m = y.shape[1]

    d = (x[:, 0:1] - y[0:1, :]) ** 2
    d = d + (x[:, 1:2] - y[1:2, :]) ** 2
    d = d + (x[:, 2:3] - y[2:3, :]) ** 2

    big = jnp.float32(2**24)

    m1 = jnp.min(d, axis=1, keepdims=True)
    jcol = lax.broadcasted_iota(jnp.int32, (1, m), 1).astype(jnp.float32)
    i1f = jnp.min(jnp.where(d == m1, jcol, big), axis=1, keepdims=True)
    d1_ref[0] = m1
    i1_ref[0] = i1f.astype(jnp.int32)

    m2 = jnp.min(d, axis=0, keepdims=True)
    riota = (lax.broadcasted_iota(jnp.int32, (x.shape[0], 1), 0)
             + ni * NB).astype(jnp.float32)
    i2f = jnp.min(jnp.where(d == m2, riota, big), axis=0, keepdims=True)
    i2 = i2f.astype(jnp.int32)

    @pl.when(ni == 0)
    def _init():
        d2_ref[0] = m2
        i2_ref[0] = i2

    @pl.when(ni != 0)
    def _acc():
        prev_d = d2_ref[0]
        prev_i = i2_ref[0]
        upd = m2 < prev_d
        d2_ref[0] = jnp.where(upd, m2, prev_d)
        i2_ref[0] = jnp.where(upd, i2, prev_i)


def kernel(input1, input2):
    b, n, _ = input1.shape
    m = input2.shape[1]
    nblk = n // NB
    y_t = input2.transpose(0, 2, 1)

    d1, i1, d2, i2 = pl.pallas_call(
        _chamfer_kernel,
        grid=(b, nblk),
        in_specs=[
            pl.BlockSpec((1, NB, 3), lambda bi, ni: (bi, ni, 0)),
            pl.BlockSpec((1, 3, m), lambda bi, ni: (bi, 0, 0)),
        ],
        out_specs=[
            pl.BlockSpec((1, NB, 1), lambda bi, ni: (bi * nblk + ni, 0, 0)),
            pl.BlockSpec((1, NB, 1), lambda bi, ni: (bi * nblk + ni, 0, 0)),
            pl.BlockSpec((1, 1, m), lambda bi, ni: (bi, 0, 0)),
            pl.BlockSpec((1, 1, m), lambda bi, ni: (bi, 0, 0)),
        ],
        out_shape=[
            jax.ShapeDtypeStruct((b * nblk, NB, 1), jnp.float32),
            jax.ShapeDtypeStruct((b * nblk, NB, 1), jnp.int32),
            jax.ShapeDtypeStruct((b, 1, m), jnp.float32),
            jax.ShapeDtypeStruct((b, 1, m), jnp.int32),
        ],
        compiler_params=pltpu.CompilerParams(
            dimension_semantics=("parallel", "arbitrary")),
    )(input1, y_t)

    dist1 = d1.reshape(b, n)
    idx1 = i1.reshape(b, n)
    dist2 = d2.reshape(b, m)
    idx2 = i2.reshape(b, m)
    return (dist1, dist2, idx1, idx2)


# speedup vs baseline: 1.0824x; 1.0173x over previous
import jax
import jax.numpy as jnp
from jax import lax
from jax.experimental import pallas as pl
from jax.experimental.pallas import tpu as pltpu

NB = 1024


def _chamfer_kernel(x_ref, y_ref, d1_ref, i1_ref, d2_ref, i2_ref):
    ni = pl.program_id(1)
    x = x_ref[0]
    y = y_ref[0]
    m = y.shape[1]

    d = (x[:, 0:1] - y[0:1, :]) ** 2
    d = d + (x[:, 1:2] - y[1:2, :]) ** 2
    d = d + (x[:, 2:3] - y[2:3, :]) ** 2

    big = jnp.float32(2**24)

    m1 = jnp.min(d, axis=1, keepdims=True)
    jcol = lax.broadcasted_iota(jnp.int32, (1, m), 1).astype(jnp.float32)
    i1f = jnp.min(jnp.where(d == m1, jcol, big), axis=1, keepdims=True)
    d1_ref[0] = m1
    i1_ref[0] = i1f.astype(jnp.int32)

    m2 = jnp.min(d, axis=0, keepdims=True)
    riota = (lax.broadcasted_iota(jnp.int32, (x.shape[0], 1), 0)
             + ni * NB).astype(jnp.float32)
    i2f = jnp.min(jnp.where(d == m2, riota, big), axis=0, keepdims=True)
    i2 = i2f.astype(jnp.int32)

    @pl.when(ni == 0)
    def _init():
        d2_ref[0] = m2
        i2_ref[0] = i2

    @pl.when(ni != 0)
    def _acc():
        prev_d = d2_ref[0]
        prev_i = i2_ref[0]
        upd = m2 < prev_d
        d2_ref[0] = jnp.where(upd, m2, prev_d)
        i2_ref[0] = jnp.where(upd, i2, prev_i)


def kernel(input1, input2):
    b, n, _ = input1.shape
    m = input2.shape[1]
    nblk = n // NB
    y_t = input2.transpose(0, 2, 1)

    d1, i1, d2, i2 = pl.pallas_call(
        _chamfer_kernel,
        grid=(b, nblk),
        in_specs=[
            pl.BlockSpec((1, NB, 3), lambda bi, ni: (bi, ni, 0)),
            pl.BlockSpec((1, 3, m), lambda bi, ni: (bi, 0, 0)),
        ],
        out_specs=[
            pl.BlockSpec((1, NB, 1), lambda bi, ni: (bi * nblk + ni, 0, 0)),
            pl.BlockSpec((1, NB, 1), lambda bi, ni: (bi * nblk + ni, 0, 0)),
            pl.BlockSpec((1, 1, m), lambda bi, ni: (bi, 0, 0)),
            pl.BlockSpec((1, 1, m), lambda bi, ni: (bi, 0, 0)),
        ],
        out_shape=[
            jax.ShapeDtypeStruct((b * nblk, NB, 1), jnp.float32),
            jax.ShapeDtypeStruct((b * nblk, NB, 1), jnp.int32),
            jax.ShapeDtypeStruct((b, 1, m), jnp.float32),
            jax.ShapeDtypeStruct((b, 1, m), jnp.int32),
        ],
        compiler_params=pltpu.CompilerParams(
            dimension_semantics=("parallel", "arbitrary")),
    )(input1, y_t)

    dist1 = d1.reshape(b, n)
    idx1 = i1.reshape(b, n)
    dist2 = d2.reshape(b, m)
    idx2 = i2.reshape(b, m)
    return (dist1, dist2, idx1, idx2)


# speedup vs baseline: 1.1156x; 1.0307x over previous
import jax
import jax.numpy as jnp
from jax import lax
from jax.experimental import pallas as pl
from jax.experimental.pallas import tpu as pltpu

NB = 2048


def _chamfer_kernel(x_ref, y_ref, d1_ref, i1_ref, d2_ref, i2_ref):
    ni = pl.program_id(1)
    x = x_ref[0]
    y = y_ref[0]
    m = y.shape[1]

    d = (x[:, 0:1] - y[0:1, :]) ** 2
    d = d + (x[:, 1:2] - y[1:2, :]) ** 2
    d = d + (x[:, 2:3] - y[2:3, :]) ** 2

    big = jnp.float32(2**24)

    m1 = jnp.min(d, axis=1, keepdims=True)
    jcol = lax.broadcasted_iota(jnp.int32, (1, m), 1).astype(jnp.float32)
    i1f = jnp.min(jnp.where(d == m1, jcol, big), axis=1, keepdims=True)
    d1_ref[0] = m1
    i1_ref[0] = i1f.astype(jnp.int32)

    m2 = jnp.min(d, axis=0, keepdims=True)
    riota = (lax.broadcasted_iota(jnp.int32, (x.shape[0], 1), 0)
             + ni * NB).astype(jnp.float32)
    i2f = jnp.min(jnp.where(d == m2, riota, big), axis=0, keepdims=True)
    i2 = i2f.astype(jnp.int32)

    @pl.when(ni == 0)
    def _init():
        d2_ref[0] = m2
        i2_ref[0] = i2

    @pl.when(ni != 0)
    def _acc():
        prev_d = d2_ref[0]
        prev_i = i2_ref[0]
        upd = m2 < prev_d
        d2_ref[0] = jnp.where(upd, m2, prev_d)
        i2_ref[0] = jnp.where(upd, i2, prev_i)


def kernel(input1, input2):
    b, n, _ = input1.shape
    m = input2.shape[1]
    nblk = n // NB
    y_t = input2.transpose(0, 2, 1)

    d1, i1, d2, i2 = pl.pallas_call(
        _chamfer_kernel,
        grid=(b, nblk),
        in_specs=[
            pl.BlockSpec((1, NB, 3), lambda bi, ni: (bi, ni, 0)),
            pl.BlockSpec((1, 3, m), lambda bi, ni: (bi, 0, 0)),
        ],
        out_specs=[
            pl.BlockSpec((1, NB, 1), lambda bi, ni: (bi * nblk + ni, 0, 0)),
            pl.BlockSpec((1, NB, 1), lambda bi, ni: (bi * nblk + ni, 0, 0)),
            pl.BlockSpec((1, 1, m), lambda bi, ni: (bi, 0, 0)),
            pl.BlockSpec((1, 1, m), lambda bi, ni: (bi, 0, 0)),
        ],
        out_shape=[
            jax.ShapeDtypeStruct((b * nblk, NB, 1), jnp.float32),
            jax.ShapeDtypeStruct((b * nblk, NB, 1), jnp.int32),
            jax.ShapeDtypeStruct((b, 1, m), jnp.float32),
            jax.ShapeDtypeStruct((b, 1, m), jnp.int32),
        ],
        compiler_params=pltpu.CompilerParams(
            dimension_semantics=("parallel", "arbitrary")),
    )(input1, y_t)

    dist1 = d1.reshape(b, n)
    idx1 = i1.reshape(b, n)
    dist2 = d2.reshape(b, m)
    idx2 = i2.reshape(b, m)
    return (dist1, dist2, idx1, idx2)
